# baseline (speedup 1.0000x reference)
import jax
import jax.numpy as jnp
from jax.experimental import pallas as pl
from jax.experimental.pallas import tpu as pltpu


def _body(x_ref, w_ref, ct_ref, tt_ref, o_ref):
    xt = x_ref[0, 0]
    acc = jax.lax.dot_general(
        xt, w_ref[...], (((1,), (0,)), ((), ())),
        preferred_element_type=jnp.float32,
    )
    o_ref[0] = acc + tt_ref[...] + ct_ref[0]


def kernel(x, W, bias, chan_table, time_table, alpha_c, alpha_t):
    b, ch, t_len, in_dim = x.shape
    d = W.shape[0]
    x = x.astype(jnp.bfloat16)
    wt = W.T.astype(jnp.bfloat16)
    ct3 = (alpha_c * chan_table).reshape(ch, 1, d)
    tvec = bias[None, :] + alpha_t * time_table

    out = pl.pallas_call(
        _body,
        grid=(b, ch),
        in_specs=[
            pl.BlockSpec((1, 1, t_len, in_dim), lambda i, j: (i, j, 0, 0)),
            pl.BlockSpec((in_dim, d), lambda i, j: (0, 0)),
            pl.BlockSpec((1, 1, d), lambda i, j: (j, 0, 0)),
            pl.BlockSpec((t_len, d), lambda i, j: (0, 0)),
        ],
        out_specs=pl.BlockSpec((1, t_len, d), lambda i, j: (i, 0, j)),
        out_shape=jax.ShapeDtypeStruct((b, t_len, ch * d), jnp.float32),
        compiler_params=pltpu.CompilerParams(
            dimension_semantics=("parallel", "parallel"),
        ),
    )(x, wt, ct3, tvec)
    return out.reshape(b, t_len * ch, d)


# speedup vs baseline: 1.1029x; 1.1029x over previous
import jax
import jax.numpy as jnp
from jax.experimental import pallas as pl
from jax.experimental.pallas import tpu as pltpu


def _body(x_ref, w_ref, ct_ref, tt_ref, o_ref):
    xt = x_ref[0, 0]
    acc = jax.lax.dot_general(
        xt, w_ref[...], (((1,), (0,)), ((), ())),
        preferred_element_type=jnp.float32,
    )
    o_ref[0] = acc + tt_ref[...] + ct_ref[0]


def kernel(x, W, bias, chan_table, time_table, alpha_c, alpha_t):
    b, ch, t_len, in_dim = x.shape
    d = W.shape[0]
    wt = W.T
    ct3 = (alpha_c * chan_table).reshape(ch, 1, d)
    tvec = bias[None, :] + alpha_t * time_table

    out = pl.pallas_call(
        _body,
        grid=(b, ch),
        in_specs=[
            pl.BlockSpec((1, 1, t_len, in_dim), lambda i, j: (i, j, 0, 0)),
            pl.BlockSpec((in_dim, d), lambda i, j: (0, 0)),
            pl.BlockSpec((1, 1, d), lambda i, j: (j, 0, 0)),
            pl.BlockSpec((t_len, d), lambda i, j: (0, 0)),
        ],
        out_specs=pl.BlockSpec((1, t_len, d), lambda i, j: (i, 0, j)),
        out_shape=jax.ShapeDtypeStruct((b, t_len, ch * d), jnp.float32),
        compiler_params=pltpu.CompilerParams(
            dimension_semantics=("parallel", "parallel"),
        ),
    )(x, wt, ct3, tvec)
    return out.reshape(b, t_len * ch, d)


# speedup vs baseline: 2.3990x; 2.1751x over previous
import jax
import jax.numpy as jnp
from jax.experimental import pallas as pl
from jax.experimental.pallas import tpu as pltpu

_TT = 8


def _body(x_ref, w_ref, ct_ref, tt_ref, o_ref):
    ch, tt_len, in_dim = x_ref.shape[1:]
    d = w_ref.shape[1]
    xt = jnp.swapaxes(x_ref[0], 0, 1)
    acc = jax.lax.dot_general(
        xt.reshape(tt_len * ch, in_dim), w_ref[...],
        (((1,), (0,)), ((), ())),
        preferred_element_type=jnp.float32,
    ).reshape(tt_len, ch, d)
    o_ref[...] = acc + tt_ref[...][:, None, :] + ct_ref[...][None, :, :]


def kernel(x, W, bias, chan_table, time_table, alpha_c, alpha_t):
    b, ch, t_len, in_dim = x.shape
    d = W.shape[0]
    wt = W.T
    ct = alpha_c * chan_table
    tvec = bias[None, :] + alpha_t * time_table
    n_t = t_len // _TT

    out = pl.pallas_call(
        _body,
        grid=(b, n_t),
        in_specs=[
            pl.BlockSpec((1, ch, _TT, in_dim), lambda i, j: (i, 0, j, 0)),
            pl.BlockSpec((in_dim, d), lambda i, j: (0, 0)),
            pl.BlockSpec((ch, d), lambda i, j: (0, 0)),
            pl.BlockSpec((_TT, d), lambda i, j: (j, 0)),
        ],
        out_specs=pl.BlockSpec((_TT, ch, d), lambda i, j: (i * n_t + j, 0, 0)),
        out_shape=jax.ShapeDtypeStruct((b * t_len, ch, d), jnp.float32),
        compiler_params=pltpu.CompilerParams(
            dimension_semantics=("parallel", "parallel"),
        ),
    )(x, wt, ct, tvec)
    return out.reshape(b, t_len * ch, d)


# speedup vs baseline: 3.0159x; 1.2571x over previous
import jax
import jax.numpy as jnp
from jax.experimental import pallas as pl
from jax.experimental.pallas import tpu as pltpu

_TT = 16


def _body(x_ref, w_ref, ct_ref, tt_ref, o_ref):
    ch, tt_len, in_dim = x_ref.shape[1:]
    d = w_ref.shape[1]
    xt = jnp.swapaxes(x_ref[0], 0, 1)
    acc = jax.lax.dot_general(
        xt.reshape(tt_len * ch, in_dim), w_ref[...],
        (((1,), (0,)), ((), ())),
        preferred_element_type=jnp.float32,
    ).reshape(tt_len, ch, d)
    o_ref[...] = acc + tt_ref[...][:, None, :] + ct_ref[...][None, :, :]


def kernel(x, W, bias, chan_table, time_table, alpha_c, alpha_t):
    b, ch, t_len, in_dim = x.shape
    d = W.shape[0]
    wt = W.T
    ct = alpha_c * chan_table
    tvec = bias[None, :] + alpha_t * time_table
    n_t = t_len // _TT

    out = pl.pallas_call(
        _body,
        grid=(b, n_t),
        in_specs=[
            pl.BlockSpec((1, ch, _TT, in_dim), lambda i, j: (i, 0, j, 0)),
            pl.BlockSpec((in_dim, d), lambda i, j: (0, 0)),
            pl.BlockSpec((ch, d), lambda i, j: (0, 0)),
            pl.BlockSpec((_TT, d), lambda i, j: (j, 0)),
        ],
        out_specs=pl.BlockSpec((_TT, ch, d), lambda i, j: (i * n_t + j, 0, 0)),
        out_shape=jax.ShapeDtypeStruct((b * t_len, ch, d), jnp.float32),
        compiler_params=pltpu.CompilerParams(
            dimension_semantics=("parallel", "parallel"),
        ),
    )(x, wt, ct, tvec)
    return out.reshape(b, t_len * ch, d)


# speedup vs baseline: 3.3725x; 1.1183x over previous
import jax
import jax.numpy as jnp
from jax.experimental import pallas as pl
from jax.experimental.pallas import tpu as pltpu

_TT = 32


def _body(x_ref, w_ref, ct_ref, tt_ref, o_ref):
    ch, tt_len, in_dim = x_ref.shape[1:]
    d = w_ref.shape[1]
    xt = jnp.swapaxes(x_ref[0], 0, 1)
    acc = jax.lax.dot_general(
        xt.reshape(tt_len * ch, in_dim), w_ref[...],
        (((1,), (0,)), ((), ())),
        preferred_element_type=jnp.float32,
    ).reshape(tt_len, ch, d)
    o_ref[...] = acc + tt_ref[...][:, None, :] + ct_ref[...][None, :, :]


def kernel(x, W, bias, chan_table, time_table, alpha_c, alpha_t):
    b, ch, t_len, in_dim = x.shape
    d = W.shape[0]
    wt = W.T
    ct = alpha_c * chan_table
    tvec = bias[None, :] + alpha_t * time_table
    n_t = t_len // _TT

    out = pl.pallas_call(
        _body,
        grid=(b, n_t),
        in_specs=[
            pl.BlockSpec((1, ch, _TT, in_dim), lambda i, j: (i, 0, j, 0)),
            pl.BlockSpec((in_dim, d), lambda i, j: (0, 0)),
            pl.BlockSpec((ch, d), lambda i, j: (0, 0)),
            pl.BlockSpec((_TT, d), lambda i, j: (j, 0)),
        ],
        out_specs=pl.BlockSpec((_TT, ch, d), lambda i, j: (i * n_t + j, 0, 0)),
        out_shape=jax.ShapeDtypeStruct((b * t_len, ch, d), jnp.float32),
        compiler_params=pltpu.CompilerParams(
            dimension_semantics=("parallel", "parallel"),
        ),
    )(x, wt, ct, tvec)
    return out.reshape(b, t_len * ch, d)


# speedup vs baseline: 3.4371x; 1.0191x over previous
import jax
import jax.numpy as jnp
from jax.experimental import pallas as pl
from jax.experimental.pallas import tpu as pltpu

_TT = 32


def _body(x_ref, w_ref, ct_ref, tt_ref, o_ref):
    ch, tt_len, in_dim = x_ref.shape[1:]
    d = w_ref.shape[1]
    xb = x_ref[0].astype(jnp.bfloat16)
    xt = jnp.swapaxes(xb, 0, 1)
    acc = jax.lax.dot_general(
        xt.reshape(tt_len * ch, in_dim), w_ref[...],
        (((1,), (0,)), ((), ())),
        preferred_element_type=jnp.float32,
    ).reshape(tt_len, ch, d)
    o_ref[...] = acc + tt_ref[...][:, None, :] + ct_ref[...][None, :, :]


def kernel(x, W, bias, chan_table, time_table, alpha_c, alpha_t):
    b, ch, t_len, in_dim = x.shape
    d = W.shape[0]
    wt = W.T.astype(jnp.bfloat16)
    ct = alpha_c * chan_table
    tvec = bias[None, :] + alpha_t * time_table
    n_t = t_len // _TT

    out = pl.pallas_call(
        _body,
        grid=(b, n_t),
        in_specs=[
            pl.BlockSpec((1, ch, _TT, in_dim), lambda i, j: (i, 0, j, 0)),
            pl.BlockSpec((in_dim, d), lambda i, j: (0, 0)),
            pl.BlockSpec((ch, d), lambda i, j: (0, 0)),
            pl.BlockSpec((_TT, d), lambda i, j: (j, 0)),
        ],
        out_specs=pl.BlockSpec((_TT, ch, d), lambda i, j: (i * n_t + j, 0, 0)),
        out_shape=jax.ShapeDtypeStruct((b * t_len, ch, d), jnp.float32),
        compiler_params=pltpu.CompilerParams(
            dimension_semantics=("parallel", "parallel"),
        ),
    )(x, wt, ct, tvec)
    return out.reshape(b, t_len * ch, d)
